# baseline (speedup 1.0000x reference)
import functools

import jax
import jax.numpy as jnp
from jax import lax
from jax.experimental import pallas as pl
from jax.experimental.pallas import tpu as pltpu
from jax.experimental.pallas import tpu_sc as plsc

_B = 1024
_T = 200
_C = 128
_NS = 16
_BB = 128
_HEAD = 128


@functools.lru_cache(maxsize=None)
def _build_sc_gather():
    mesh = plsc.VectorSubcoreMesh(
        core_axis_name="c", subcore_axis_name="s", num_cores=1)
    bpw = _B // _NS

    @functools.partial(
        pl.kernel,
        mesh=mesh,
        out_type=(
            jax.ShapeDtypeStruct((_B, _C), jnp.float32),
            jax.ShapeDtypeStruct((_B, _C), jnp.float32),
        ),
        scratch_types=[
            pltpu.VMEM((bpw,), jnp.int32),
            pltpu.VMEM((bpw, _C), jnp.float32),
            pltpu.VMEM((bpw, _C), jnp.float32),
            pltpu.SemaphoreType.DMA,
            pltpu.SemaphoreType.DMA,
        ],
    )
    def _sc_gather(idx_hbm, gamma_hbm, beta_hbm, g_out, b_out,
                   idx_v, g_rows, b_rows, sem_g, sem_b):
        wid = lax.axis_index("s")
        base = wid * bpw
        pltpu.sync_copy(idx_hbm.at[pl.ds(base, bpw)], idx_v)
        cg = pltpu.async_copy(gamma_hbm.at[idx_v], g_rows, sem_g)
        cb = pltpu.async_copy(beta_hbm.at[idx_v], b_rows, sem_b)
        cg.wait()
        cb.wait()
        pltpu.sync_copy(g_rows, g_out.at[pl.ds(base, bpw)])
        pltpu.sync_copy(b_rows, b_out.at[pl.ds(base, bpw)])

    return _sc_gather


def _head_body(ids_ref, x_ref, gamma_ref, beta_ref, o_ref, g_scr, b_scr):
    base = pl.program_id(0) * _BB

    def gather(i, carry):
        sid = ids_ref[base + i]
        g_scr[pl.ds(i, 1)] = gamma_ref[pl.ds(sid, 1), :]
        b_scr[pl.ds(i, 1)] = beta_ref[pl.ds(sid, 1), :]
        return carry

    lax.fori_loop(0, _BB, gather, 0)
    g = g_scr[...][:, None, :]
    b = b_scr[...][:, None, :]
    o_ref[...] = x_ref[...] * g + b


def _head(idx, x, gamma, beta):
    n_subj = gamma.shape[0]
    grid_spec = pltpu.PrefetchScalarGridSpec(
        num_scalar_prefetch=1,
        grid=(_HEAD // _BB,),
        in_specs=[
            pl.BlockSpec((_BB, _T, _C), lambda i, ids: (i, 0, 0)),
            pl.BlockSpec((n_subj, _C), lambda i, ids: (0, 0)),
            pl.BlockSpec((n_subj, _C), lambda i, ids: (0, 0)),
        ],
        out_specs=pl.BlockSpec((_BB, _T, _C), lambda i, ids: (i, 0, 0)),
        scratch_shapes=[
            pltpu.VMEM((_BB, _C), jnp.float32),
            pltpu.VMEM((_BB, _C), jnp.float32),
        ],
    )
    return pl.pallas_call(
        _head_body,
        grid_spec=grid_spec,
        out_shape=jax.ShapeDtypeStruct((_B, _T, _C), jnp.float32),
    )(idx, x, gamma, beta)


def _tail_body(big_ref, x_ref, g_ref, b_ref, o_ref):
    del big_ref
    g = g_ref[...][:, None, :]
    b = b_ref[...][:, None, :]
    o_ref[...] = x_ref[...] * g + b


def _tail(big, x, g, b):
    off = _HEAD // _BB
    return pl.pallas_call(
        _tail_body,
        grid=((_B - _HEAD) // _BB,),
        in_specs=[
            pl.BlockSpec(memory_space=pl.ANY),
            pl.BlockSpec((_BB, _T, _C), lambda i: (i + off, 0, 0)),
            pl.BlockSpec((_BB, _C), lambda i: (i + off, 0)),
            pl.BlockSpec((_BB, _C), lambda i: (i + off, 0)),
        ],
        out_specs=pl.BlockSpec((_BB, _T, _C), lambda i: (i + off, 0, 0)),
        out_shape=jax.ShapeDtypeStruct((_B, _T, _C), jnp.float32),
        input_output_aliases={0: 0},
    )(big, x, g, b)


def kernel(x, subject_ids, gamma, beta):
    idx = subject_ids.astype(jnp.int32)
    g, b = _build_sc_gather()(idx, gamma, beta)
    big = _head(idx, x, gamma, beta)
    return _tail(big, x, g, b)


# speedup vs baseline: 1.0279x; 1.0279x over previous
import functools

import jax
import jax.numpy as jnp
from jax import lax
from jax.experimental import pallas as pl
from jax.experimental.pallas import tpu as pltpu
from jax.experimental.pallas import tpu_sc as plsc

_B = 1024
_T = 200
_C = 128
_NS = 16
_BB = 128
_HEAD = 256


@functools.lru_cache(maxsize=None)
def _build_sc_gather():
    mesh = plsc.VectorSubcoreMesh(
        core_axis_name="c", subcore_axis_name="s", num_cores=1)
    bpw = _B // _NS

    @functools.partial(
        pl.kernel,
        mesh=mesh,
        out_type=(
            jax.ShapeDtypeStruct((_B, _C), jnp.float32),
            jax.ShapeDtypeStruct((_B, _C), jnp.float32),
        ),
        scratch_types=[
            pltpu.VMEM((bpw,), jnp.int32),
            pltpu.VMEM((bpw, _C), jnp.float32),
            pltpu.VMEM((bpw, _C), jnp.float32),
            pltpu.SemaphoreType.DMA,
            pltpu.SemaphoreType.DMA,
        ],
    )
    def _sc_gather(idx_hbm, gamma_hbm, beta_hbm, g_out, b_out,
                   idx_v, g_rows, b_rows, sem_g, sem_b):
        wid = lax.axis_index("s")
        base = wid * bpw
        pltpu.sync_copy(idx_hbm.at[pl.ds(base, bpw)], idx_v)
        cg = pltpu.async_copy(gamma_hbm.at[idx_v], g_rows, sem_g)
        cb = pltpu.async_copy(beta_hbm.at[idx_v], b_rows, sem_b)
        cg.wait()
        cb.wait()
        pltpu.sync_copy(g_rows, g_out.at[pl.ds(base, bpw)])
        pltpu.sync_copy(b_rows, b_out.at[pl.ds(base, bpw)])

    return _sc_gather


def _head_body(ids_ref, x_ref, gamma_ref, beta_ref, o_ref, g_scr, b_scr):
    base = pl.program_id(0) * _BB

    def gather(i, carry):
        sid = ids_ref[base + i]
        g_scr[pl.ds(i, 1)] = gamma_ref[pl.ds(sid, 1), :]
        b_scr[pl.ds(i, 1)] = beta_ref[pl.ds(sid, 1), :]
        return carry

    lax.fori_loop(0, _BB, gather, 0)
    g = g_scr[...][:, None, :]
    b = b_scr[...][:, None, :]
    o_ref[...] = x_ref[...] * g + b


def _head(idx, x, gamma, beta):
    n_subj = gamma.shape[0]
    grid_spec = pltpu.PrefetchScalarGridSpec(
        num_scalar_prefetch=1,
        grid=(_HEAD // _BB,),
        in_specs=[
            pl.BlockSpec((_BB, _T, _C), lambda i, ids: (i, 0, 0)),
            pl.BlockSpec((n_subj, _C), lambda i, ids: (0, 0)),
            pl.BlockSpec((n_subj, _C), lambda i, ids: (0, 0)),
        ],
        out_specs=pl.BlockSpec((_BB, _T, _C), lambda i, ids: (i, 0, 0)),
        scratch_shapes=[
            pltpu.VMEM((_BB, _C), jnp.float32),
            pltpu.VMEM((_BB, _C), jnp.float32),
        ],
    )
    return pl.pallas_call(
        _head_body,
        grid_spec=grid_spec,
        out_shape=jax.ShapeDtypeStruct((_B, _T, _C), jnp.float32),
    )(idx, x, gamma, beta)


def _tail_body(big_ref, x_ref, g_ref, b_ref, o_ref):
    del big_ref
    g = g_ref[...][:, None, :]
    b = b_ref[...][:, None, :]
    o_ref[...] = x_ref[...] * g + b


def _tail(big, x, g, b):
    off = _HEAD // _BB
    return pl.pallas_call(
        _tail_body,
        grid=((_B - _HEAD) // _BB,),
        in_specs=[
            pl.BlockSpec(memory_space=pl.ANY),
            pl.BlockSpec((_BB, _T, _C), lambda i: (i + off, 0, 0)),
            pl.BlockSpec((_BB, _C), lambda i: (i + off, 0)),
            pl.BlockSpec((_BB, _C), lambda i: (i + off, 0)),
        ],
        out_specs=pl.BlockSpec((_BB, _T, _C), lambda i: (i + off, 0, 0)),
        out_shape=jax.ShapeDtypeStruct((_B, _T, _C), jnp.float32),
        input_output_aliases={0: 0},
    )(big, x, g, b)


def kernel(x, subject_ids, gamma, beta):
    idx = subject_ids.astype(jnp.int32)
    g, b = _build_sc_gather()(idx, gamma, beta)
    big = _head(idx, x, gamma, beta)
    return _tail(big, x, g, b)
